# Initial kernel scaffold; baseline (speedup 1.0000x reference)
import jax
import jax.numpy as jnp
from jax.experimental import pallas as pl


def kernel(micro, label, micro_all, label_all, W):
    raise NotImplementedError("write your pallas kernel here")


# speedup vs baseline: 1.9312x; 1.9312x over previous
import functools

import jax
import jax.numpy as jnp
from jax import lax
from jax.experimental import pallas as pl
from jax.experimental.pallas import tpu as pltpu
from jax.experimental.pallas import tpu_sc as plsc

Q, K, D = 128, 2048, 256
KNN = 6
NEG_INF = float("-inf")

QB = 8
KB = 512


def _logits_tc_kernel(micro_ref, micro_all_ref, wb_ref, out_ref):
    m = micro_ref[...]
    a = micro_all_ref[...]
    wb = wb_ref[...].astype(jnp.bfloat16).astype(jnp.float32)
    diff = a[None, :, :] - m[:, None, :]
    dis = diff * diff
    mean2 = jnp.sum(dis, axis=2) * jnp.float32(1.0 / D)
    disb = dis.astype(jnp.bfloat16).astype(jnp.float32)
    term2 = jnp.sum(disb * wb[None, :, :], axis=2)
    out_ref[...] = term2 - mean2


def _run_logits_tc(micro, micro_all, wb):
    return pl.pallas_call(
        _logits_tc_kernel,
        grid=(K // KB, Q // QB),
        in_specs=[
            pl.BlockSpec((QB, D), lambda kb, qb: (qb, 0)),
            pl.BlockSpec((KB, D), lambda kb, qb: (kb, 0)),
            pl.BlockSpec((1, D), lambda kb, qb: (0, 0)),
        ],
        out_specs=pl.BlockSpec((QB, KB), lambda kb, qb: (qb, kb)),
        out_shape=jax.ShapeDtypeStruct((Q, K), jnp.float32),
    )(micro, micro_all, wb)


def _topk_tc_kernel(logits_ref, label_ref, label_all_ref,
                    w_out_ref, idx_out_ref, closs_ref):
    iota_k = lax.broadcasted_iota(jnp.int32, (Q, K), 1)
    work = logits_ref[...]
    vals = []
    idxs = []
    for _ in range(KNN):
        m = jnp.max(work, axis=1, keepdims=True)
        idx_j = jnp.max(jnp.where(work == m, iota_k, -1),
                        axis=1, keepdims=True)
        onehot = iota_k == idx_j
        work = jnp.where(onehot, NEG_INF, work)
        vals.append(m)
        idxs.append(idx_j)

    m0 = vals[0]
    es = [jnp.exp(vj - m0) for vj in vals]
    denom = es[0]
    for e in es[1:]:
        denom = denom + e
    ws = [e / denom for e in es]

    w_out_ref[...] = jnp.concatenate(
        ws + [jnp.zeros((Q, 2), jnp.float32)], axis=1)
    idx_out_ref[...] = jnp.concatenate(
        idxs + [jnp.zeros((Q, 2), jnp.int32)], axis=1)

    dis_label = jnp.abs(label_all_ref[...] - label_ref[...])
    cut = jnp.zeros((Q, K), jnp.float32)
    for w_j, idx_j in zip(ws, idxs):
        cut = cut + jnp.where(iota_k == idx_j, w_j, 0.0)
    closs_ref[...] = (jnp.float32(1e-4)
                      + jnp.sum(cut * dis_label, keepdims=True) / Q)


def _run_topk_tc(logits, label, label_all):
    return pl.pallas_call(
        _topk_tc_kernel,
        out_shape=(
            jax.ShapeDtypeStruct((Q, 8), jnp.float32),
            jax.ShapeDtypeStruct((Q, 8), jnp.int32),
            jax.ShapeDtypeStruct((1, 1), jnp.float32),
        ),
    )(logits, label, label_all)


_NC, _NS = 2, 16
_NW = _NC * _NS
_QPW = Q // _NW
_IPW = _QPW * KNN
_DC = D // 16


def _round_bf16(x):
    xi = lax.bitcast_convert_type(x, jnp.int32)
    rounded = (xi + 0x7FFF + ((xi >> 16) & 1)) & jnp.int32(-65536)
    return lax.bitcast_convert_type(rounded, jnp.float32)


def _gather_sc_kernel(micro_hbm, table_hbm, idx_hbm, wb_hbm, out_hbm,
                      idx_v, rows_v, wv, mic_v, out_v, sem):
    wid = lax.axis_index("s") * _NC + lax.axis_index("c")
    qbase = wid * _QPW
    ibase = wid * _IPW
    pltpu.sync_copy(idx_hbm.at[pl.ds(ibase, _IPW)], idx_v)
    pltpu.async_copy(table_hbm.at[idx_v], rows_v, sem).wait()
    pltpu.sync_copy(wb_hbm.at[pl.ds(ibase, _IPW)], wv)
    pltpu.sync_copy(micro_hbm.at[pl.ds(qbase, _QPW)], mic_v)
    for qi in range(_QPW):
        wvecs = [_round_bf16(wv[qi * KNN + j, :]) for j in range(KNN)]
        for c in range(_DC):
            acc = mic_v[qi, pl.ds(c * 16, 16)]
            for j in range(KNN):
                acc = acc + wvecs[j] * _round_bf16(
                    rows_v[qi * KNN + j, pl.ds(c * 16, 16)])
            out_v[qi, pl.ds(c * 16, 16)] = acc
    pltpu.sync_copy(out_v, out_hbm.at[pl.ds(qbase, _QPW)])


@functools.cache
def _build_gather_sc():
    return functools.partial(
        pl.kernel,
        mesh=plsc.VectorSubcoreMesh(core_axis_name="c", subcore_axis_name="s"),
        out_type=jax.ShapeDtypeStruct((Q, D), jnp.float32),
        scratch_types=[
            pltpu.VMEM((_IPW,), jnp.int32),
            pltpu.VMEM((_IPW, D), jnp.float32),
            pltpu.VMEM((_IPW, 16), jnp.float32),
            pltpu.VMEM((_QPW, D), jnp.float32),
            pltpu.VMEM((_QPW, D), jnp.float32),
            pltpu.SemaphoreType.DMA,
        ],
    )(_gather_sc_kernel)


def kernel(micro, label, micro_all, label_all, W):
    logits = _run_logits_tc(micro, micro_all, W)
    w8, idx8, closs = _run_topk_tc(
        logits, label.reshape(Q, 1), label_all.reshape(1, K))
    idx_flat = idx8[:, :KNN].reshape(-1)
    wbr = jnp.broadcast_to(w8[:, :KNN].reshape(-1)[:, None],
                           (Q * KNN, 16))
    micro_tmp = _build_gather_sc()(micro, micro_all, idx_flat, wbr)
    return micro_tmp, closs[0, 0]
